# baseline (speedup 1.0000x reference)
import functools

import jax
import jax.numpy as jnp
from jax import lax
from jax.experimental import pallas as pl
from jax.experimental.pallas import tpu as pltpu
from jax.experimental.pallas import tpu_sc as plsc

T = 576
B = 64
C = 768
KEEP = 144

NC = 2
NS = 16
NW = NC * NS

N_FWD = T * B
N_VIS = KEEP * B
VIS_PER_W = N_VIS // NW
BATCH_PER_W = B // NW
COL_PER_W = BATCH_PER_W * T

CHUNK = 48
N_CHUNK = VIS_PER_W // CHUNK
NBUF = 2


def _body(patches_hbm, fwd_hbm, vis_hbm, bwd_hbm,
          fwdv, col, fwd_b, gidx, bufs, sem_f, sem_b, sem_s, sem_g, wsems):
    wid = lax.axis_index("s") * NC + lax.axis_index("c")
    lane = lax.iota(jnp.int32, 16)

    fload = pltpu.async_copy(fwd_hbm, fwdv, sem_f)
    pltpu.async_copy(
        fwd_hbm.at[pl.ds(wid * VIS_PER_W, VIS_PER_W)], fwd_b, sem_b).wait()

    def body_b(j, carry):
        e = wid * VIS_PER_W + j * 16 + lane
        f = fwd_b[pl.ds(j * 16, 16)]
        gidx[pl.ds(j * 16, 16)] = f * B + e % B
        return carry

    lax.fori_loop(0, VIS_PER_W // 16, body_b, 0)

    def gather(k):
        return pltpu.async_copy(
            patches_hbm.at[gidx.at[pl.ds(k * CHUNK, CHUNK)]],
            bufs[k % NBUF], sem_g)

    def write(k):
        return pltpu.async_copy(
            bufs[k % NBUF],
            vis_hbm.at[pl.ds(wid * VIS_PER_W + k * CHUNK, CHUNK)],
            wsems[k % NBUF])

    gh = [None] * N_CHUNK
    wh = [None] * N_CHUNK
    for k in range(NBUF):
        gh[k] = gather(k)

    fload.wait()

    def body_a(j, carry):
        e = j * 16 + lane
        i_vec = e % T
        par = e // T
        b_vec = wid * BATCH_PER_W + par
        f = plsc.load_gather(fwdv, [i_vec * B + b_vec])
        plsc.store_scatter(col, [par * T + f], i_vec)
        return carry

    lax.fori_loop(0, COL_PER_W // 16, body_a, 0)
    scat = pltpu.async_copy(
        col, bwd_hbm.at[pl.ds(wid * COL_PER_W, COL_PER_W)], sem_s)

    for k in range(N_CHUNK):
        gh[k].wait()
        if k >= 1:
            wh[k - 1].wait()
            if k - 1 + NBUF < N_CHUNK:
                gh[k - 1 + NBUF] = gather(k - 1 + NBUF)
        wh[k] = write(k)
    wh[N_CHUNK - 1].wait()
    scat.wait()


@functools.partial(
    pl.kernel,
    out_type=[
        jax.ShapeDtypeStruct((N_VIS, C), jnp.float32),
        jax.ShapeDtypeStruct((N_FWD,), jnp.int32),
    ],
    mesh=plsc.VectorSubcoreMesh(core_axis_name="c", subcore_axis_name="s"),
    compiler_params=pltpu.CompilerParams(needs_layout_passes=False),
    scratch_types=[
        pltpu.VMEM((N_FWD,), jnp.int32),
        pltpu.VMEM((COL_PER_W,), jnp.int32),
        pltpu.VMEM((VIS_PER_W,), jnp.int32),
        pltpu.VMEM((VIS_PER_W,), jnp.int32),
        [pltpu.VMEM((CHUNK, C), jnp.float32) for _ in range(NBUF)],
        pltpu.SemaphoreType.DMA,
        pltpu.SemaphoreType.DMA,
        pltpu.SemaphoreType.DMA,
        pltpu.SemaphoreType.DMA,
        [pltpu.SemaphoreType.DMA for _ in range(NBUF)],
    ],
)
def _patch_shuffle(patches_hbm, fwd_hbm, vis_hbm, bwd_hbm, *rest):
    _body(patches_hbm, fwd_hbm, vis_hbm, bwd_hbm, *rest)


def kernel(patches, forward_indexes):
    p_flat = patches.reshape(T * B, C)
    f_flat = forward_indexes.reshape(N_FWD)
    vis_flat, bwd_t = _patch_shuffle(p_flat, f_flat)
    return (vis_flat.reshape(KEEP, B, C), forward_indexes,
            bwd_t.reshape(B, T).T)


# speedup vs baseline: 1.0251x; 1.0251x over previous
import functools

import jax
import jax.numpy as jnp
from jax import lax
from jax.experimental import pallas as pl
from jax.experimental.pallas import tpu as pltpu
from jax.experimental.pallas import tpu_sc as plsc

T = 576
B = 64
C = 768
KEEP = 144

NC = 2
NS = 16
NW = NC * NS

N_FWD = T * B
N_VIS = KEEP * B
VIS_PER_W = N_VIS // NW
BATCH_PER_W = B // NW
COL_PER_W = BATCH_PER_W * T

CHUNK = 16
N_CHUNK = VIS_PER_W // CHUNK
NBUF = 6


def _body(patches_hbm, fwd_hbm, vis_hbm, bwd_hbm,
          fwdv, col, fwd_b, gidx, bufs, sem_f, sem_b, sem_s, sem_g, wsems):
    wid = lax.axis_index("s") * NC + lax.axis_index("c")
    lane = lax.iota(jnp.int32, 16)

    fload = pltpu.async_copy(fwd_hbm, fwdv, sem_f)
    pltpu.async_copy(
        fwd_hbm.at[pl.ds(wid * VIS_PER_W, VIS_PER_W)], fwd_b, sem_b).wait()

    def body_b(j, carry):
        e = wid * VIS_PER_W + j * 16 + lane
        f = fwd_b[pl.ds(j * 16, 16)]
        gidx[pl.ds(j * 16, 16)] = f * B + e % B
        return carry

    lax.fori_loop(0, VIS_PER_W // 16, body_b, 0)

    def gather(k):
        return pltpu.async_copy(
            patches_hbm.at[gidx.at[pl.ds(k * CHUNK, CHUNK)]],
            bufs[k % NBUF], sem_g)

    def write(k):
        return pltpu.async_copy(
            bufs[k % NBUF],
            vis_hbm.at[pl.ds(wid * VIS_PER_W + k * CHUNK, CHUNK)],
            wsems[k % NBUF])

    gh = [None] * N_CHUNK
    wh = [None] * N_CHUNK
    for k in range(NBUF):
        gh[k] = gather(k)

    fload.wait()

    def body_a(j, carry):
        e = j * 16 + lane
        i_vec = e % T
        par = e // T
        b_vec = wid * BATCH_PER_W + par
        f = plsc.load_gather(fwdv, [i_vec * B + b_vec])
        plsc.store_scatter(col, [par * T + f], i_vec)
        return carry

    lax.fori_loop(0, COL_PER_W // 16, body_a, 0)
    scat = pltpu.async_copy(
        col, bwd_hbm.at[pl.ds(wid * COL_PER_W, COL_PER_W)], sem_s)

    for k in range(N_CHUNK):
        gh[k].wait()
        if k >= 1:
            wh[k - 1].wait()
            if k - 1 + NBUF < N_CHUNK:
                gh[k - 1 + NBUF] = gather(k - 1 + NBUF)
        wh[k] = write(k)
    wh[N_CHUNK - 1].wait()
    scat.wait()


@functools.partial(
    pl.kernel,
    out_type=[
        jax.ShapeDtypeStruct((N_VIS, C), jnp.float32),
        jax.ShapeDtypeStruct((N_FWD,), jnp.int32),
    ],
    mesh=plsc.VectorSubcoreMesh(core_axis_name="c", subcore_axis_name="s"),
    compiler_params=pltpu.CompilerParams(needs_layout_passes=False),
    scratch_types=[
        pltpu.VMEM((N_FWD,), jnp.int32),
        pltpu.VMEM((COL_PER_W,), jnp.int32),
        pltpu.VMEM((VIS_PER_W,), jnp.int32),
        pltpu.VMEM((VIS_PER_W,), jnp.int32),
        [pltpu.VMEM((CHUNK, C), jnp.float32) for _ in range(NBUF)],
        pltpu.SemaphoreType.DMA,
        pltpu.SemaphoreType.DMA,
        pltpu.SemaphoreType.DMA,
        pltpu.SemaphoreType.DMA,
        [pltpu.SemaphoreType.DMA for _ in range(NBUF)],
    ],
)
def _patch_shuffle(patches_hbm, fwd_hbm, vis_hbm, bwd_hbm, *rest):
    _body(patches_hbm, fwd_hbm, vis_hbm, bwd_hbm, *rest)


def kernel(patches, forward_indexes):
    p_flat = patches.reshape(T * B, C)
    f_flat = forward_indexes.reshape(N_FWD)
    vis_flat, bwd_t = _patch_shuffle(p_flat, f_flat)
    return (vis_flat.reshape(KEEP, B, C), forward_indexes,
            bwd_t.reshape(B, T).T)


# speedup vs baseline: 1.0305x; 1.0053x over previous
import functools

import jax
import jax.numpy as jnp
from jax import lax
from jax.experimental import pallas as pl
from jax.experimental.pallas import tpu as pltpu
from jax.experimental.pallas import tpu_sc as plsc

T = 576
B = 64
C = 768
KEEP = 144

NC = 2
NS = 16
NW = NC * NS

N_FWD = T * B
N_VIS = KEEP * B
VIS_PER_W = N_VIS // NW
BATCH_PER_W = B // NW
COL_PER_W = BATCH_PER_W * T

CHUNK = 24
N_CHUNK = VIS_PER_W // CHUNK
NBUF = 4
N_OUTER = N_CHUNK // NBUF


def _body(patches_hbm, fwd_hbm, vis_hbm, bwd_hbm,
          fwdv, col, fwd_b, gidx, bufs, sem_f, sem_b, sem_s, sem_g, wsems):
    wid = lax.axis_index("s") * NC + lax.axis_index("c")
    lane = lax.iota(jnp.int32, 16)

    fload = pltpu.async_copy(fwd_hbm, fwdv, sem_f)
    pltpu.async_copy(
        fwd_hbm.at[pl.ds(wid * VIS_PER_W, VIS_PER_W)], fwd_b, sem_b).wait()

    def body_b(j, carry):
        e = wid * VIS_PER_W + j * 16 + lane
        f = fwd_b[pl.ds(j * 16, 16)]
        gidx[pl.ds(j * 16, 16)] = f * B + e % B
        return carry

    lax.fori_loop(0, VIS_PER_W // 16, body_b, 0)

    def gather(k, r):
        return pltpu.async_copy(
            patches_hbm.at[gidx.at[pl.ds(k * CHUNK, CHUNK)]],
            bufs[r], sem_g)

    def write(k, r):
        return pltpu.async_copy(
            bufs[r],
            vis_hbm.at[pl.ds(wid * VIS_PER_W + k * CHUNK, CHUNK)],
            wsems[r])

    for r in range(NBUF):
        gather(r, r)

    fload.wait()

    def body_a(j, carry):
        e = j * 16 + lane
        i_vec = e % T
        par = e // T
        b_vec = wid * BATCH_PER_W + par
        f = plsc.load_gather(fwdv, [i_vec * B + b_vec])
        plsc.store_scatter(col, [par * T + f], i_vec)
        return carry

    lax.fori_loop(0, COL_PER_W // 16, body_a, 0)
    scat = pltpu.async_copy(
        col, bwd_hbm.at[pl.ds(wid * COL_PER_W, COL_PER_W)], sem_s)

    def wait_gather(r):
        pltpu.make_async_copy(
            patches_hbm.at[pl.ds(0, CHUNK)], bufs[r], sem_g).wait()

    def wait_write(r):
        pltpu.make_async_copy(
            bufs[r], vis_hbm.at[pl.ds(0, CHUNK)], wsems[r]).wait()

    def outer(o, carry):
        for r in range(NBUF):
            k = o * NBUF + r
            wait_gather(r)
            rp = (r - 1) % NBUF

            @pl.when(k >= 1)
            def _():
                wait_write(rp)

            @pl.when(jnp.logical_and(k >= 1, k - 1 + NBUF < N_CHUNK))
            def _():
                gather(k - 1 + NBUF, rp)

            write(k, r)
        return carry

    lax.fori_loop(0, N_OUTER, outer, 0)
    wait_write((N_CHUNK - 1) % NBUF)
    scat.wait()


@functools.partial(
    pl.kernel,
    out_type=[
        jax.ShapeDtypeStruct((N_VIS, C), jnp.float32),
        jax.ShapeDtypeStruct((N_FWD,), jnp.int32),
    ],
    mesh=plsc.VectorSubcoreMesh(core_axis_name="c", subcore_axis_name="s"),
    compiler_params=pltpu.CompilerParams(needs_layout_passes=False),
    scratch_types=[
        pltpu.VMEM((N_FWD,), jnp.int32),
        pltpu.VMEM((COL_PER_W,), jnp.int32),
        pltpu.VMEM((VIS_PER_W,), jnp.int32),
        pltpu.VMEM((VIS_PER_W,), jnp.int32),
        [pltpu.VMEM((CHUNK, C), jnp.float32) for _ in range(NBUF)],
        pltpu.SemaphoreType.DMA,
        pltpu.SemaphoreType.DMA,
        pltpu.SemaphoreType.DMA,
        pltpu.SemaphoreType.DMA,
        [pltpu.SemaphoreType.DMA for _ in range(NBUF)],
    ],
)
def _patch_shuffle(patches_hbm, fwd_hbm, vis_hbm, bwd_hbm, *rest):
    _body(patches_hbm, fwd_hbm, vis_hbm, bwd_hbm, *rest)


def kernel(patches, forward_indexes):
    p_flat = patches.reshape(T * B, C)
    f_flat = forward_indexes.reshape(N_FWD)
    vis_flat, bwd_t = _patch_shuffle(p_flat, f_flat)
    return (vis_flat.reshape(KEEP, B, C), forward_indexes,
            bwd_t.reshape(B, T).T)
